# Initial kernel scaffold; baseline (speedup 1.0000x reference)
import jax
import jax.numpy as jnp
from jax.experimental import pallas as pl


def kernel(word, position1, position2, W_word, W_pos1, W_pos2):
    raise NotImplementedError("write your pallas kernel here")


# speedup vs baseline: 1.9475x; 1.9475x over previous
import jax
import jax.numpy as jnp
from jax import lax
from jax.experimental import pallas as pl
from jax.experimental.pallas import tpu as pltpu
from jax.experimental.pallas import tpu_sc as plsc

BAG = 1024
SEQ = 200
WORD_DIM = 64
POS_DIM = 16
OUT_DIM = WORD_DIM + 2 * POS_DIM
N_TOK = BAG * SEQ
NUM_CORES = 2
NUM_SUBCORES = 16
NW = NUM_CORES * NUM_SUBCORES
PER_W = N_TOK // NW
IDX_MINOR = 128
CHUNK = 256
N_SUB = CHUNK // IDX_MINOR
N_CHUNKS = PER_W // CHUNK
IDX_ROWS_PER_W = PER_W // IDX_MINOR


def _body(word_hbm, pos1_hbm, pos2_hbm, ww_hbm, wp1_hbm, wp2_hbm, out_hbm,
          widx, p1idx, p2idx, wrows, p1rows, p2rows, sem):
    wid = lax.axis_index("s") * NUM_CORES + lax.axis_index("c")
    base_row = wid * IDX_ROWS_PER_W

    @pl.loop(0, N_CHUNKS)
    def _chunk(g):
        r0 = base_row + g * N_SUB
        pltpu.sync_copy(word_hbm.at[pl.ds(r0, N_SUB)], widx)
        pltpu.sync_copy(pos1_hbm.at[pl.ds(r0, N_SUB)], p1idx)
        pltpu.sync_copy(pos2_hbm.at[pl.ds(r0, N_SUB)], p2idx)
        cps = []
        for j in range(N_SUB):
            dst = pl.ds(j * IDX_MINOR, IDX_MINOR)
            cps.append(pltpu.async_copy(ww_hbm.at[widx.at[j]], wrows.at[dst], sem))
            cps.append(pltpu.async_copy(wp1_hbm.at[p1idx.at[j]], p1rows.at[dst], sem))
            cps.append(pltpu.async_copy(wp2_hbm.at[p2idx.at[j]], p2rows.at[dst], sem))
        for cp in cps:
            cp.wait()
        tok0 = wid * PER_W + g * CHUNK
        rows = pl.ds(tok0, CHUNK)
        pltpu.sync_copy(wrows, out_hbm.at[rows, pl.ds(0, WORD_DIM)])
        pltpu.sync_copy(p1rows, out_hbm.at[rows, pl.ds(WORD_DIM, POS_DIM)])
        pltpu.sync_copy(p2rows, out_hbm.at[rows, pl.ds(WORD_DIM + POS_DIM, POS_DIM)])


_embed = pl.kernel(
    _body,
    out_type=jax.ShapeDtypeStruct((N_TOK, OUT_DIM), jnp.float32),
    mesh=plsc.VectorSubcoreMesh(core_axis_name="c", subcore_axis_name="s"),
    scratch_types=[
        pltpu.VMEM((N_SUB, IDX_MINOR), jnp.int32),
        pltpu.VMEM((N_SUB, IDX_MINOR), jnp.int32),
        pltpu.VMEM((N_SUB, IDX_MINOR), jnp.int32),
        pltpu.VMEM((CHUNK, WORD_DIM), jnp.float32),
        pltpu.VMEM((CHUNK, POS_DIM), jnp.float32),
        pltpu.VMEM((CHUNK, POS_DIM), jnp.float32),
        pltpu.SemaphoreType.DMA,
    ],
    compiler_params=pltpu.CompilerParams(use_tc_tiling_on_sc=False),
)


def kernel(word, position1, position2, W_word, W_pos1, W_pos2):
    word2 = word.reshape(-1, IDX_MINOR)
    pos1_2 = position1.reshape(-1, IDX_MINOR)
    pos2_2 = position2.reshape(-1, IDX_MINOR)
    out = _embed(word2, pos1_2, pos2_2, W_word, W_pos1, W_pos2)
    return out.reshape(BAG, 1, SEQ, OUT_DIM)


# speedup vs baseline: 2.0270x; 1.0408x over previous
import jax
import jax.numpy as jnp
from jax import lax
from jax.experimental import pallas as pl
from jax.experimental.pallas import tpu as pltpu
from jax.experimental.pallas import tpu_sc as plsc

BAG = 1024
SEQ = 200
WORD_DIM = 64
POS_DIM = 16
OUT_DIM = WORD_DIM + 2 * POS_DIM
N_TOK = BAG * SEQ
NUM_CORES = 2
NUM_SUBCORES = 16
NW = NUM_CORES * NUM_SUBCORES
PER_W = N_TOK // NW
CHUNK = 128
N_CHUNKS = PER_W // CHUNK
NSLOT = 8
GDIST = 4
TOTAL_STEPS = ((N_CHUNKS + GDIST + NSLOT - 1) // NSLOT) * NSLOT

P1_OFF = WORD_DIM
P2_OFF = WORD_DIM + POS_DIM


def _body(word_hbm, pos1_hbm, pos2_hbm, ww_hbm, wp1_hbm, wp2_hbm, out_hbm,
          widx, p1idx, p2idx, wrows, p1rows, p2rows, gsems, wsems):
    wid = lax.axis_index("s") * NUM_CORES + lax.axis_index("c")
    base_row = wid * N_CHUNKS
    tok_base = wid * PER_W

    def out_slices(g):
        rows = pl.ds(tok_base + g * CHUNK, CHUNK)
        return (out_hbm.at[rows, pl.ds(0, WORD_DIM)],
                out_hbm.at[rows, pl.ds(P1_OFF, POS_DIM)],
                out_hbm.at[rows, pl.ds(P2_OFF, POS_DIM)])

    def slot_bufs(b):
        sl = pl.ds(b * CHUNK, CHUNK)
        return wrows.at[sl], p1rows.at[sl], p2rows.at[sl]

    def wait_writes(b, g):
        wr, p1r, p2r = slot_bufs(b)
        ow, o1, o2 = out_slices(g)
        pltpu.make_async_copy(wr, ow, wsems.at[b]).wait()
        pltpu.make_async_copy(p1r, o1, wsems.at[b]).wait()
        pltpu.make_async_copy(p2r, o2, wsems.at[b]).wait()

    @pl.loop(0, TOTAL_STEPS, step=NSLOT)
    def _steps(g0):
        for b in range(NSLOT):
            g = g0 + b
            gd = g - GDIST
            bd = (b - GDIST) % NSLOT

            @pl.when((gd >= 0) & (gd < N_CHUNKS))
            def _drain():
                wr, p1r, p2r = slot_bufs(bd)
                ow, o1, o2 = out_slices(gd)
                pltpu.make_async_copy(ww_hbm.at[widx.at[bd]], wr, gsems.at[bd]).wait()
                pltpu.make_async_copy(wp1_hbm.at[p1idx.at[bd]], p1r, gsems.at[bd]).wait()
                pltpu.make_async_copy(wp2_hbm.at[p2idx.at[bd]], p2r, gsems.at[bd]).wait()
                pltpu.async_copy(wr, ow, wsems.at[bd])
                pltpu.async_copy(p1r, o1, wsems.at[bd])
                pltpu.async_copy(p2r, o2, wsems.at[bd])

            @pl.when(g < N_CHUNKS)
            def _fire():
                @pl.when(g >= NSLOT)
                def _wait_prev_write():
                    wait_writes(b, g - NSLOT)

                wr, p1r, p2r = slot_bufs(b)
                r0 = base_row + g
                pltpu.sync_copy(word_hbm.at[pl.ds(r0, 1)], widx.at[pl.ds(b, 1)])
                pltpu.sync_copy(pos1_hbm.at[pl.ds(r0, 1)], p1idx.at[pl.ds(b, 1)])
                pltpu.sync_copy(pos2_hbm.at[pl.ds(r0, 1)], p2idx.at[pl.ds(b, 1)])
                pltpu.async_copy(ww_hbm.at[widx.at[b]], wr, gsems.at[b])
                pltpu.async_copy(wp1_hbm.at[p1idx.at[b]], p1r, gsems.at[b])
                pltpu.async_copy(wp2_hbm.at[p2idx.at[b]], p2r, gsems.at[b])

    for g in range(N_CHUNKS - NSLOT, N_CHUNKS):
        wait_writes(g % NSLOT, g)


_embed = pl.kernel(
    _body,
    out_type=jax.ShapeDtypeStruct((N_TOK, OUT_DIM), jnp.float32),
    mesh=plsc.VectorSubcoreMesh(core_axis_name="c", subcore_axis_name="s"),
    scratch_types=[
        pltpu.VMEM((NSLOT, CHUNK), jnp.int32),
        pltpu.VMEM((NSLOT, CHUNK), jnp.int32),
        pltpu.VMEM((NSLOT, CHUNK), jnp.int32),
        pltpu.VMEM((NSLOT * CHUNK, WORD_DIM), jnp.float32),
        pltpu.VMEM((NSLOT * CHUNK, POS_DIM), jnp.float32),
        pltpu.VMEM((NSLOT * CHUNK, POS_DIM), jnp.float32),
        pltpu.SemaphoreType.DMA((NSLOT,)),
        pltpu.SemaphoreType.DMA((NSLOT,)),
    ],
    compiler_params=pltpu.CompilerParams(use_tc_tiling_on_sc=False),
)


def kernel(word, position1, position2, W_word, W_pos1, W_pos2):
    word2 = word.reshape(-1, CHUNK)
    pos1_2 = position1.reshape(-1, CHUNK)
    pos2_2 = position2.reshape(-1, CHUNK)
    out = _embed(word2, pos1_2, pos2_2, W_word, W_pos1, W_pos2)
    return out.reshape(BAG, 1, SEQ, OUT_DIM)


# speedup vs baseline: 2.0355x; 1.0042x over previous
import jax
import jax.numpy as jnp
from jax import lax
from jax.experimental import pallas as pl
from jax.experimental.pallas import tpu as pltpu
from jax.experimental.pallas import tpu_sc as plsc

BAG = 1024
SEQ = 200
WORD_DIM = 64
POS_DIM = 16
OUT_DIM = WORD_DIM + 2 * POS_DIM
NUM_CORES = 2
NUM_SUBCORES = 16
NW = NUM_CORES * NUM_SUBCORES
BAGS_PER_W = BAG // NW
SPLIT = 128
REM = SEQ - SPLIT
N_CHUNKS = 2 * BAGS_PER_W
NSLOT = 8
GDIST = 4
TOTAL_STEPS = ((N_CHUNKS + GDIST + NSLOT - 1) // NSLOT) * NSLOT

P1_OFF = WORD_DIM
P2_OFF = WORD_DIM + POS_DIM


def _body(word_hbm, pos1_hbm, pos2_hbm, ww_hbm, wp1_hbm, wp2_hbm, out_hbm,
          widx, p1idx, p2idx, wrows, p1rows, p2rows, gsems, wsems):
    wid = lax.axis_index("s") * NUM_CORES + lax.axis_index("c")
    bag0 = wid * BAGS_PER_W

    pltpu.sync_copy(word_hbm.at[pl.ds(bag0, BAGS_PER_W)], widx)
    pltpu.sync_copy(pos1_hbm.at[pl.ds(bag0, BAGS_PER_W)], p1idx)
    pltpu.sync_copy(pos2_hbm.at[pl.ds(bag0, BAGS_PER_W)], p2idx)

    def chunk_len(b):
        return SPLIT if b % 2 == 0 else REM

    def slot_bufs(b):
        sl = pl.ds(b * SPLIT, chunk_len(b))
        return wrows.at[sl], p1rows.at[sl], p2rows.at[sl]

    def out_slices(c, b):
        gbag = bag0 + lax.div(c, 2)
        t0 = 0 if b % 2 == 0 else SPLIT
        toks = pl.ds(t0, chunk_len(b))
        return (out_hbm.at[gbag, 0, toks, pl.ds(0, WORD_DIM)],
                out_hbm.at[gbag, 0, toks, pl.ds(P1_OFF, POS_DIM)],
                out_hbm.at[gbag, 0, toks, pl.ds(P2_OFF, POS_DIM)])

    def idx_slices(c, b):
        i = lax.div(c, 2)
        t0 = 0 if b % 2 == 0 else SPLIT
        sl = pl.ds(t0, chunk_len(b))
        return widx.at[i, sl], p1idx.at[i, sl], p2idx.at[i, sl]

    def wait_writes(b, c):
        wr, p1r, p2r = slot_bufs(b)
        ow, o1, o2 = out_slices(c, b)
        pltpu.make_async_copy(wr, ow, wsems.at[b]).wait()
        pltpu.make_async_copy(p1r, o1, wsems.at[b]).wait()
        pltpu.make_async_copy(p2r, o2, wsems.at[b]).wait()

    @pl.loop(0, TOTAL_STEPS, step=NSLOT)
    def _steps(c0):
        for b in range(NSLOT):
            c = c0 + b
            cd = c - GDIST
            bd = (b - GDIST) % NSLOT

            @pl.when((cd >= 0) & (cd < N_CHUNKS))
            def _drain():
                wr, p1r, p2r = slot_bufs(bd)
                iw, i1, i2 = idx_slices(cd, bd)
                ow, o1, o2 = out_slices(cd, bd)
                pltpu.make_async_copy(ww_hbm.at[iw], wr, gsems.at[bd]).wait()
                pltpu.make_async_copy(wp1_hbm.at[i1], p1r, gsems.at[bd]).wait()
                pltpu.make_async_copy(wp2_hbm.at[i2], p2r, gsems.at[bd]).wait()
                pltpu.async_copy(wr, ow, wsems.at[bd])
                pltpu.async_copy(p1r, o1, wsems.at[bd])
                pltpu.async_copy(p2r, o2, wsems.at[bd])

            @pl.when(c < N_CHUNKS)
            def _fire():
                @pl.when(c >= NSLOT)
                def _wait_prev_write():
                    wait_writes(b, c - NSLOT)

                wr, p1r, p2r = slot_bufs(b)
                iw, i1, i2 = idx_slices(c, b)
                pltpu.async_copy(ww_hbm.at[iw], wr, gsems.at[b])
                pltpu.async_copy(wp1_hbm.at[i1], p1r, gsems.at[b])
                pltpu.async_copy(wp2_hbm.at[i2], p2r, gsems.at[b])

    for c in range(N_CHUNKS - NSLOT, N_CHUNKS):
        wait_writes(c % NSLOT, c)


_embed = pl.kernel(
    _body,
    out_type=jax.ShapeDtypeStruct((BAG, 1, SEQ, OUT_DIM), jnp.float32),
    mesh=plsc.VectorSubcoreMesh(core_axis_name="c", subcore_axis_name="s"),
    scratch_types=[
        pltpu.VMEM((BAGS_PER_W, SEQ), jnp.int32),
        pltpu.VMEM((BAGS_PER_W, SEQ), jnp.int32),
        pltpu.VMEM((BAGS_PER_W, SEQ), jnp.int32),
        pltpu.VMEM((NSLOT * SPLIT, WORD_DIM), jnp.float32),
        pltpu.VMEM((NSLOT * SPLIT, POS_DIM), jnp.float32),
        pltpu.VMEM((NSLOT * SPLIT, POS_DIM), jnp.float32),
        pltpu.SemaphoreType.DMA((NSLOT,)),
        pltpu.SemaphoreType.DMA((NSLOT,)),
    ],
    compiler_params=pltpu.CompilerParams(use_tc_tiling_on_sc=False),
)


def kernel(word, position1, position2, W_word, W_pos1, W_pos2):
    return _embed(word, position1, position2, W_word, W_pos1, W_pos2)
